# baseline (speedup 1.0000x reference)
import functools

import numpy as np
import jax
import jax.numpy as jnp
from jax.experimental import pallas as pl
from jax.experimental.pallas import tpu as pltpu
from jax.experimental.pallas import tpu_sc as plsc

D_MODEL = 512
D_K = 64
D_V = 64
H = 8
_C = 5


def _threefry2x32_np(k1, k2, x1, x2):
    u32 = np.uint32
    def rotl(v, d):
        return ((v << u32(d)) | (v >> u32(32 - d))).astype(u32)
    ks = [u32(k1), u32(k2), u32(k1) ^ u32(k2) ^ u32(0x1BD11BDA)]
    x = [(x1 + ks[0]).astype(u32), (x2 + ks[1]).astype(u32)]
    rotations = ((13, 15, 26, 6), (17, 29, 16, 24))
    for i in range(5):
        for r in rotations[i % 2]:
            x[0] = (x[0] + x[1]).astype(u32)
            x[1] = x[0] ^ rotl(x[1], r)
        x[0] = (x[0] + ks[(i + 1) % 3]).astype(u32)
        x[1] = (x[1] + ks[(i + 2) % 3] + u32(i + 1)).astype(u32)
    return x[0], x[1]


def _randint_np(seed, shape, minval, maxval):
    u32 = np.uint32
    n = int(np.prod(shape))
    b1, b2 = _threefry2x32_np(u32(0), u32(seed),
                              np.zeros(2, u32), np.arange(2, dtype=u32))
    idx = np.arange(n, dtype=np.uint64)
    hi = (idx >> np.uint64(32)).astype(u32)
    lo = idx.astype(u32)

    def rbits(ka, kb):
        a, b = _threefry2x32_np(ka, kb, hi, lo)
        return a ^ b

    higher_bits = rbits(b1[0], b2[0])
    lower_bits = rbits(b1[1], b2[1])
    span = u32(maxval - minval)
    multiplier = u32((2 ** 16) % int(span))
    multiplier = u32((int(multiplier) * int(multiplier)) % int(span))
    with np.errstate(over='ignore'):
        offset = ((higher_bits % span) * multiplier + (lower_bits % span)) % span
    return (np.int32(minval) + offset.astype(np.int32)).reshape(shape)


@functools.lru_cache(maxsize=None)
def _constants(L_Q, L_K):
    u_k = min(int(_C * np.log(L_K)), L_Q)
    u_q = min(int(_C * np.log(L_Q)), L_Q)
    idx = _randint_np(42, (L_Q, u_k), 0, L_K)
    cnt = np.zeros((L_Q, L_K), np.float32)
    np.add.at(cnt, (np.arange(L_Q)[:, None], idx), 1.0)
    tri = np.tril(np.ones((L_K, L_K), np.float32))
    return u_k, u_q, cnt, tri


def _stage_a_kernel(u_k, L_Q, L_K,
                    xq_ref, xk_ref, xv_ref, wq_ref, wk_ref, wv_ref,
                    cnt_ref, tri_ref, m_ref, vals_ref, csum_ref):
    f32 = jnp.float32
    neg = f32(-jnp.inf)
    xq = xq_ref[0]
    xk = xk_ref[0]
    xv = xv_ref[0]
    cnt = cnt_ref[...]
    supported = cnt > 0
    tri = tri_ref[...]

    for h in range(H):
        Qh = jnp.dot(xq, wq_ref[h], preferred_element_type=f32)
        Kh = jnp.dot(xk, wk_ref[h], preferred_element_type=f32)
        Vh = jnp.dot(xv, wv_ref[h], preferred_element_type=f32)
        S = jnp.dot(Qh, Kh.T, preferred_element_type=f32)
        m_max = jnp.max(jnp.where(supported, S, neg), axis=1, keepdims=True)
        m_mean = jnp.sum(S * cnt, axis=1, keepdims=True) * f32(1.0 / u_k)
        m_ref[h:h + 1, :] = (m_max - m_mean).T
        Ss = S * f32(1.0 / np.sqrt(D_K))
        e = jnp.exp(Ss - jnp.max(Ss, axis=1, keepdims=True))
        r = f32(1.0) / jnp.sum(e, axis=1, keepdims=True)
        vals_ref[0, h] = jnp.dot(e, Vh, preferred_element_type=f32) * r
        csum_ref[0, h] = jnp.dot(tri, Vh, preferred_element_type=f32)


def _make_sc_topk(u_q, R, L):
    info = plsc.get_sparse_core_info()
    nc = info.num_cores
    mesh = plsc.VectorSubcoreMesh(core_axis_name="c", subcore_axis_name="s")

    nchunk = L // 16

    @functools.partial(
        pl.kernel, mesh=mesh,
        out_type=jax.ShapeDtypeStruct((R, L), jnp.float32),
        scratch_types=[
            pltpu.VMEM((L,), jnp.float32),
            pltpu.VMEM((L,), jnp.float32),
        ],
    )
    def sc_topk(m_hbm, sel_hbm, row_v, sel_v):
        wid = jax.lax.axis_index("s") * nc + jax.lax.axis_index("c")

        @pl.when(wid < R)
        def _():
            pltpu.sync_copy(m_hbm.at[wid], row_v)
            neg = jnp.float32(-jnp.inf)
            iota = jax.lax.iota(jnp.int32, 16)
            perms = [iota ^ k for k in (1, 2, 4, 8)]

            def body(_, carry):
                best = row_v[pl.ds(0, 16)]
                for i in range(1, nchunk):
                    best = jnp.maximum(best, row_v[pl.ds(i * 16, 16)])
                dnums = jax.lax.GatherDimensionNumbers(
                    offset_dims=(), collapsed_slice_dims=(0,),
                    start_index_map=(0,))
                for p in perms:
                    shuf = jax.lax.gather(
                        best, p[:, None], dnums, slice_sizes=(1,),
                        mode=jax.lax.GatherScatterMode.PROMISE_IN_BOUNDS)
                    best = jnp.maximum(best, shuf)
                for i in range(nchunk):
                    sl = pl.ds(i * 16, 16)
                    chunk = row_v[sl]
                    row_v[sl] = jnp.where(chunk == best, neg, chunk)
                return carry

            jax.lax.fori_loop(0, u_q, body, jnp.int32(0))
            one = jnp.full((16,), 1.0, jnp.float32)
            zero = jnp.zeros((16,), jnp.float32)
            for i in range(nchunk):
                sl = pl.ds(i * 16, 16)
                sel_v[sl] = jnp.where(row_v[sl] == neg, one, zero)
            pltpu.sync_copy(sel_v, sel_hbm.at[wid])

    return sc_topk


def _stage_b_kernel(sel_ref, vals_ref, csum_ref, xq_ref, wfc_ref,
                    g_ref, b_ref, o_ref):
    f32 = jnp.float32
    acc = None
    for h in range(H):
        sel_h = sel_ref[h:h + 1].T > f32(0.5)
        ctx = jnp.where(sel_h, vals_ref[0, h], csum_ref[0, h])
        part = jnp.dot(ctx, wfc_ref[h], preferred_element_type=f32)
        acc = part if acc is None else acc + part
    x = acc + xq_ref[0]
    mu = jnp.mean(x, axis=1, keepdims=True)
    xc = x - mu
    var = jnp.mean(xc * xc, axis=1, keepdims=True)
    o_ref[0] = xc * jax.lax.rsqrt(var + f32(1e-5)) * g_ref[...] + b_ref[...]


def _one_batch(u_k, u_q, L_Q, L_K, cnt, tri, sc_topk,
               input_Q, input_K, input_V, wq, wk, wv, wfc, g, b):
    B = 1
    m_all, vals, csum = pl.pallas_call(
        functools.partial(_stage_a_kernel, u_k, L_Q, L_K),
        grid=(B,),
        in_specs=[
            pl.BlockSpec((1, L_Q, D_MODEL), lambda b: (b, 0, 0)),
            pl.BlockSpec((1, L_K, D_MODEL), lambda b: (b, 0, 0)),
            pl.BlockSpec((1, L_K, D_MODEL), lambda b: (b, 0, 0)),
            pl.BlockSpec((H, D_MODEL, D_K), lambda b: (0, 0, 0)),
            pl.BlockSpec((H, D_MODEL, D_K), lambda b: (0, 0, 0)),
            pl.BlockSpec((H, D_MODEL, D_V), lambda b: (0, 0, 0)),
            pl.BlockSpec((L_Q, L_K), lambda b: (0, 0)),
            pl.BlockSpec((L_K, L_K), lambda b: (0, 0)),
        ],
        out_specs=[
            pl.BlockSpec((H, L_Q), lambda b: (b, 0)),
            pl.BlockSpec((1, H, L_Q, D_V), lambda b: (b, 0, 0, 0)),
            pl.BlockSpec((1, H, L_K, D_V), lambda b: (b, 0, 0, 0)),
        ],
        out_shape=[
            jax.ShapeDtypeStruct((B * H, L_Q), jnp.float32),
            jax.ShapeDtypeStruct((B, H, L_Q, D_V), jnp.float32),
            jax.ShapeDtypeStruct((B, H, L_K, D_V), jnp.float32),
        ],
        compiler_params=pltpu.CompilerParams(
            dimension_semantics=("parallel",)),
    )(input_Q, input_K, input_V, wq, wk, wv, cnt, tri)

    sel = sc_topk(m_all)

    out = pl.pallas_call(
        _stage_b_kernel,
        grid=(B,),
        in_specs=[
            pl.BlockSpec((H, L_Q), lambda b: (b, 0)),
            pl.BlockSpec((1, H, L_Q, D_V), lambda b: (b, 0, 0, 0)),
            pl.BlockSpec((1, H, L_K, D_V), lambda b: (b, 0, 0, 0)),
            pl.BlockSpec((1, L_Q, D_MODEL), lambda b: (b, 0, 0)),
            pl.BlockSpec((H, D_V, D_MODEL), lambda b: (0, 0, 0)),
            pl.BlockSpec((1, D_MODEL), lambda b: (0, 0)),
            pl.BlockSpec((1, D_MODEL), lambda b: (0, 0)),
        ],
        out_specs=pl.BlockSpec((1, L_Q, D_MODEL), lambda b: (b, 0, 0)),
        out_shape=jax.ShapeDtypeStruct((B, L_Q, D_MODEL), jnp.float32),
        compiler_params=pltpu.CompilerParams(
            dimension_semantics=("parallel",)),
    )(sel, vals, csum, input_Q, wfc, g, b)
    return out


def kernel(input_Q, input_K, input_V, attn_mask, W_Q, W_K, W_V, W_fc,
           ln_gamma, ln_beta):
    B, L_Q, _ = input_Q.shape
    L_K = input_K.shape[1]
    u_k, u_q, cnt_np, tri_np = _constants(L_Q, L_K)
    cnt = jnp.asarray(cnt_np)
    tri = jnp.asarray(tri_np)
    wq = W_Q.reshape(D_MODEL, H, D_K).transpose(1, 0, 2)
    wk = W_K.reshape(D_MODEL, H, D_K).transpose(1, 0, 2)
    wv = W_V.reshape(D_MODEL, H, D_V).transpose(1, 0, 2)
    wfc = W_fc.reshape(H, D_V, D_MODEL)
    g = ln_gamma.reshape(1, D_MODEL)
    be = ln_beta.reshape(1, D_MODEL)
    sc_topk = _make_sc_topk(u_q, H, L_Q)

    outs = [
        _one_batch(u_k, u_q, L_Q, L_K, cnt, tri, sc_topk,
                   input_Q[i:i + 1], input_K[i:i + 1], input_V[i:i + 1],
                   wq, wk, wv, wfc, g, be)
        for i in range(B)
    ]
    return jnp.concatenate(outs, axis=0)


# speedup vs baseline: 1.2310x; 1.2310x over previous
import functools

import numpy as np
import jax
import jax.numpy as jnp
from jax.experimental import pallas as pl
from jax.experimental.pallas import tpu as pltpu
from jax.experimental.pallas import tpu_sc as plsc

D_MODEL = 512
D_K = 64
D_V = 64
H = 8
_C = 5


def _threefry2x32_np(k1, k2, x1, x2):
    u32 = np.uint32
    def rotl(v, d):
        return ((v << u32(d)) | (v >> u32(32 - d))).astype(u32)
    ks = [u32(k1), u32(k2), u32(k1) ^ u32(k2) ^ u32(0x1BD11BDA)]
    x = [(x1 + ks[0]).astype(u32), (x2 + ks[1]).astype(u32)]
    rotations = ((13, 15, 26, 6), (17, 29, 16, 24))
    for i in range(5):
        for r in rotations[i % 2]:
            x[0] = (x[0] + x[1]).astype(u32)
            x[1] = x[0] ^ rotl(x[1], r)
        x[0] = (x[0] + ks[(i + 1) % 3]).astype(u32)
        x[1] = (x[1] + ks[(i + 2) % 3] + u32(i + 1)).astype(u32)
    return x[0], x[1]


def _randint_np(seed, shape, minval, maxval):
    u32 = np.uint32
    n = int(np.prod(shape))
    b1, b2 = _threefry2x32_np(u32(0), u32(seed),
                              np.zeros(2, u32), np.arange(2, dtype=u32))
    idx = np.arange(n, dtype=np.uint64)
    hi = (idx >> np.uint64(32)).astype(u32)
    lo = idx.astype(u32)

    def rbits(ka, kb):
        a, b = _threefry2x32_np(ka, kb, hi, lo)
        return a ^ b

    higher_bits = rbits(b1[0], b2[0])
    lower_bits = rbits(b1[1], b2[1])
    span = u32(maxval - minval)
    multiplier = u32((2 ** 16) % int(span))
    multiplier = u32((int(multiplier) * int(multiplier)) % int(span))
    with np.errstate(over='ignore'):
        offset = ((higher_bits % span) * multiplier + (lower_bits % span)) % span
    return (np.int32(minval) + offset.astype(np.int32)).reshape(shape)


@functools.lru_cache(maxsize=None)
def _constants(L_Q, L_K):
    u_k = min(int(_C * np.log(L_K)), L_Q)
    u_q = min(int(_C * np.log(L_Q)), L_Q)
    idx = _randint_np(42, (L_Q, u_k), 0, L_K)
    cnt = np.zeros((L_Q, L_K), np.float32)
    np.add.at(cnt, (np.arange(L_Q)[:, None], idx), 1.0)
    tri = np.tril(np.ones((L_K, L_K), np.float32))
    return u_k, u_q, cnt, tri


def _stage_a_kernel(u_k, L_Q, L_K,
                    xq_ref, xk_ref, xv_ref, wq_ref, wk_ref, wv_ref,
                    cnt_ref, tri_ref, m_ref, vals_ref, csum_ref):
    f32 = jnp.float32
    neg = f32(-jnp.inf)
    xq = xq_ref[0]
    xk = xk_ref[0]
    xv = xv_ref[0]
    cnt = cnt_ref[...]
    supported = cnt > 0
    tri = tri_ref[...]

    for h in range(H):
        Qh = jnp.dot(xq, wq_ref[h], preferred_element_type=f32)
        Kh = jnp.dot(xk, wk_ref[h], preferred_element_type=f32)
        Vh = jnp.dot(xv, wv_ref[h], preferred_element_type=f32)
        S = jnp.dot(Qh, Kh.T, preferred_element_type=f32)
        m_max = jnp.max(jnp.where(supported, S, neg), axis=1, keepdims=True)
        m_mean = jnp.sum(S * cnt, axis=1, keepdims=True) * f32(1.0 / u_k)
        m_ref[h:h + 1, :] = (m_max - m_mean).T
        Ss = S * f32(1.0 / np.sqrt(D_K))
        e = jnp.exp(Ss - jnp.max(Ss, axis=1, keepdims=True))
        r = f32(1.0) / jnp.sum(e, axis=1, keepdims=True)
        vals_ref[0, h] = jnp.dot(e, Vh, preferred_element_type=f32) * r
        csum_ref[0, h] = jnp.dot(tri, Vh, preferred_element_type=f32)


def _make_sc_topk(u_q, R, L):
    info = plsc.get_sparse_core_info()
    nc = info.num_cores
    mesh = plsc.VectorSubcoreMesh(core_axis_name="c", subcore_axis_name="s")

    nchunk = L // 16

    @functools.partial(
        pl.kernel, mesh=mesh,
        out_type=jax.ShapeDtypeStruct((R, L), jnp.float32),
        scratch_types=[
            pltpu.VMEM((L,), jnp.float32),
            pltpu.VMEM((L,), jnp.float32),
        ],
    )
    def sc_topk(m_hbm, sel_hbm, row_v, sel_v):
        wid = jax.lax.axis_index("s") * nc + jax.lax.axis_index("c")

        @pl.when(wid < R)
        def _():
            pltpu.sync_copy(m_hbm.at[wid], row_v)
            neg = jnp.float32(-jnp.inf)
            iota = jax.lax.iota(jnp.int32, 16)
            perms = [iota ^ k for k in (1, 2, 4, 8)]

            def body(_, carry):
                best = row_v[pl.ds(0, 16)]
                for i in range(1, nchunk):
                    best = jnp.maximum(best, row_v[pl.ds(i * 16, 16)])
                dnums = jax.lax.GatherDimensionNumbers(
                    offset_dims=(), collapsed_slice_dims=(0,),
                    start_index_map=(0,))
                for p in perms:
                    shuf = jax.lax.gather(
                        best, p[:, None], dnums, slice_sizes=(1,),
                        mode=jax.lax.GatherScatterMode.PROMISE_IN_BOUNDS)
                    best = jnp.maximum(best, shuf)
                for i in range(nchunk):
                    sl = pl.ds(i * 16, 16)
                    chunk = row_v[sl]
                    row_v[sl] = jnp.where(chunk == best, neg, chunk)
                return carry

            jax.lax.fori_loop(0, u_q, body, jnp.int32(0))
            one = jnp.full((16,), 1.0, jnp.float32)
            zero = jnp.zeros((16,), jnp.float32)
            for i in range(nchunk):
                sl = pl.ds(i * 16, 16)
                sel_v[sl] = jnp.where(row_v[sl] == neg, one, zero)
            pltpu.sync_copy(sel_v, sel_hbm.at[wid])

    return sc_topk


def _stage_b_kernel(sel_ref, vals_ref, csum_ref, xq_ref, wfc_ref,
                    g_ref, b_ref, o_ref):
    f32 = jnp.float32
    acc = None
    for h in range(H):
        sel_h = sel_ref[h:h + 1].T > f32(0.5)
        ctx = jnp.where(sel_h, vals_ref[0, h], csum_ref[0, h])
        part = jnp.dot(ctx, wfc_ref[h], preferred_element_type=f32)
        acc = part if acc is None else acc + part
    x = acc + xq_ref[0]
    mu = jnp.mean(x, axis=1, keepdims=True)
    xc = x - mu
    var = jnp.mean(xc * xc, axis=1, keepdims=True)
    o_ref[0] = xc * jax.lax.rsqrt(var + f32(1e-5)) * g_ref[...] + b_ref[...]


def kernel(input_Q, input_K, input_V, attn_mask, W_Q, W_K, W_V, W_fc,
           ln_gamma, ln_beta):
    B, L_Q, _ = input_Q.shape
    L_K = input_K.shape[1]
    u_k, u_q, cnt_np, tri_np = _constants(L_Q, L_K)
    cnt = jnp.asarray(cnt_np)
    tri = jnp.asarray(tri_np)

    m_all, vals, csum = pl.pallas_call(
        functools.partial(_stage_a_kernel, u_k, L_Q, L_K),
        grid=(B,),
        in_specs=[
            pl.BlockSpec((1, L_Q, D_MODEL), lambda b: (b, 0, 0)),
            pl.BlockSpec((1, L_K, D_MODEL), lambda b: (b, 0, 0)),
            pl.BlockSpec((1, L_K, D_MODEL), lambda b: (b, 0, 0)),
            pl.BlockSpec((H, D_MODEL, D_K), lambda b: (0, 0, 0)),
            pl.BlockSpec((H, D_MODEL, D_K), lambda b: (0, 0, 0)),
            pl.BlockSpec((H, D_MODEL, D_V), lambda b: (0, 0, 0)),
            pl.BlockSpec((L_Q, L_K), lambda b: (0, 0)),
            pl.BlockSpec((L_K, L_K), lambda b: (0, 0)),
        ],
        out_specs=[
            pl.BlockSpec((H, L_Q), lambda b: (b, 0)),
            pl.BlockSpec((1, H, L_Q, D_V), lambda b: (b, 0, 0, 0)),
            pl.BlockSpec((1, H, L_K, D_V), lambda b: (b, 0, 0, 0)),
        ],
        out_shape=[
            jax.ShapeDtypeStruct((B * H, L_Q), jnp.float32),
            jax.ShapeDtypeStruct((B, H, L_Q, D_V), jnp.float32),
            jax.ShapeDtypeStruct((B, H, L_K, D_V), jnp.float32),
        ],
        compiler_params=pltpu.CompilerParams(
            dimension_semantics=("parallel",)),
    )(input_Q, input_K, input_V,
      W_Q.reshape(D_MODEL, H, D_K).transpose(1, 0, 2),
      W_K.reshape(D_MODEL, H, D_K).transpose(1, 0, 2),
      W_V.reshape(D_MODEL, H, D_V).transpose(1, 0, 2),
      cnt, tri)

    sel = _make_sc_topk(u_q, B * H, L_Q)(m_all)

    out = pl.pallas_call(
        _stage_b_kernel,
        grid=(B,),
        in_specs=[
            pl.BlockSpec((H, L_Q), lambda b: (b, 0)),
            pl.BlockSpec((1, H, L_Q, D_V), lambda b: (b, 0, 0, 0)),
            pl.BlockSpec((1, H, L_K, D_V), lambda b: (b, 0, 0, 0)),
            pl.BlockSpec((1, L_Q, D_MODEL), lambda b: (b, 0, 0)),
            pl.BlockSpec((H, D_V, D_MODEL), lambda b: (0, 0, 0)),
            pl.BlockSpec((1, D_MODEL), lambda b: (0, 0)),
            pl.BlockSpec((1, D_MODEL), lambda b: (0, 0)),
        ],
        out_specs=pl.BlockSpec((1, L_Q, D_MODEL), lambda b: (b, 0, 0)),
        out_shape=jax.ShapeDtypeStruct((B, L_Q, D_MODEL), jnp.float32),
        compiler_params=pltpu.CompilerParams(
            dimension_semantics=("parallel",)),
    )(sel, vals, csum, input_Q, W_fc.reshape(H, D_V, D_MODEL),
      ln_gamma.reshape(1, D_MODEL), ln_beta.reshape(1, D_MODEL))
    return out


# speedup vs baseline: 1.2361x; 1.0041x over previous
import functools

import numpy as np
import jax
import jax.numpy as jnp
from jax.experimental import pallas as pl
from jax.experimental.pallas import tpu as pltpu
from jax.experimental.pallas import tpu_sc as plsc

D_MODEL = 512
D_K = 64
D_V = 64
H = 8
_C = 5


def _threefry2x32_np(k1, k2, x1, x2):
    u32 = np.uint32
    def rotl(v, d):
        return ((v << u32(d)) | (v >> u32(32 - d))).astype(u32)
    ks = [u32(k1), u32(k2), u32(k1) ^ u32(k2) ^ u32(0x1BD11BDA)]
    x = [(x1 + ks[0]).astype(u32), (x2 + ks[1]).astype(u32)]
    rotations = ((13, 15, 26, 6), (17, 29, 16, 24))
    for i in range(5):
        for r in rotations[i % 2]:
            x[0] = (x[0] + x[1]).astype(u32)
            x[1] = x[0] ^ rotl(x[1], r)
        x[0] = (x[0] + ks[(i + 1) % 3]).astype(u32)
        x[1] = (x[1] + ks[(i + 2) % 3] + u32(i + 1)).astype(u32)
    return x[0], x[1]


def _randint_np(seed, shape, minval, maxval):
    u32 = np.uint32
    n = int(np.prod(shape))
    b1, b2 = _threefry2x32_np(u32(0), u32(seed),
                              np.zeros(2, u32), np.arange(2, dtype=u32))
    idx = np.arange(n, dtype=np.uint64)
    hi = (idx >> np.uint64(32)).astype(u32)
    lo = idx.astype(u32)

    def rbits(ka, kb):
        a, b = _threefry2x32_np(ka, kb, hi, lo)
        return a ^ b

    higher_bits = rbits(b1[0], b2[0])
    lower_bits = rbits(b1[1], b2[1])
    span = u32(maxval - minval)
    multiplier = u32((2 ** 16) % int(span))
    multiplier = u32((int(multiplier) * int(multiplier)) % int(span))
    with np.errstate(over='ignore'):
        offset = ((higher_bits % span) * multiplier + (lower_bits % span)) % span
    return (np.int32(minval) + offset.astype(np.int32)).reshape(shape)


@functools.lru_cache(maxsize=None)
def _constants(L_Q, L_K):
    u_k = min(int(_C * np.log(L_K)), L_Q)
    u_q = min(int(_C * np.log(L_Q)), L_Q)
    idx = _randint_np(42, (L_Q, u_k), 0, L_K)
    cnt = np.zeros((L_Q, L_K), np.float32)
    np.add.at(cnt, (np.arange(L_Q)[:, None], idx), 1.0)
    tri = np.tril(np.ones((L_K, L_K), np.float32))
    return u_k, u_q, cnt, tri


def _stage_a_kernel(u_k, L_Q, L_K,
                    xq_ref, xk_ref, xv_ref, wq_ref, wk_ref, wv_ref,
                    cnt_ref, tri_ref, m_ref, vals_ref, csum_ref):
    f32 = jnp.float32
    neg = f32(-jnp.inf)
    xq = xq_ref[0]
    xk = xk_ref[0]
    xv = xv_ref[0]
    cnt = cnt_ref[...].astype(f32)
    supported = cnt > f32(0)
    tri = tri_ref[...].astype(f32)

    for h in range(H):
        Qh = jnp.dot(xq, wq_ref[h], preferred_element_type=f32)
        Kh = jnp.dot(xk, wk_ref[h], preferred_element_type=f32)
        Vh = jnp.dot(xv, wv_ref[h], preferred_element_type=f32)
        S = jnp.dot(Qh, Kh.T, preferred_element_type=f32)
        m_max = jnp.max(jnp.where(supported, S, neg), axis=1, keepdims=True)
        m_mean = jnp.sum(S * cnt, axis=1, keepdims=True) * f32(1.0 / u_k)
        m_ref[h:h + 1, :] = (m_max - m_mean).T
        Ss = S * f32(1.0 / np.sqrt(D_K))
        e = jnp.exp(Ss - jnp.max(Ss, axis=1, keepdims=True))
        r = f32(1.0) / jnp.sum(e, axis=1, keepdims=True)
        vals_ref[0, h] = jnp.dot(e, Vh, preferred_element_type=f32) * r
        csum_ref[0, h] = jnp.dot(tri, Vh, preferred_element_type=f32)


def _make_sc_topk(u_q, R, L):
    info = plsc.get_sparse_core_info()
    nc = info.num_cores
    mesh = plsc.VectorSubcoreMesh(core_axis_name="c", subcore_axis_name="s")

    nchunk = L // 16

    @functools.partial(
        pl.kernel, mesh=mesh,
        out_type=jax.ShapeDtypeStruct((R, L), jnp.float32),
        scratch_types=[
            pltpu.VMEM((L,), jnp.float32),
            pltpu.VMEM((L,), jnp.float32),
        ],
    )
    def sc_topk(m_hbm, sel_hbm, row_v, sel_v):
        wid = jax.lax.axis_index("s") * nc + jax.lax.axis_index("c")

        @pl.when(wid < R)
        def _():
            pltpu.sync_copy(m_hbm.at[wid], row_v)
            neg = jnp.float32(-jnp.inf)
            iota = jax.lax.iota(jnp.int32, 16)
            perms = [iota ^ k for k in (1, 2, 4, 8)]

            def body(_, carry):
                best = row_v[pl.ds(0, 16)]
                for i in range(1, nchunk):
                    best = jnp.maximum(best, row_v[pl.ds(i * 16, 16)])
                dnums = jax.lax.GatherDimensionNumbers(
                    offset_dims=(), collapsed_slice_dims=(0,),
                    start_index_map=(0,))
                for p in perms:
                    shuf = jax.lax.gather(
                        best, p[:, None], dnums, slice_sizes=(1,),
                        mode=jax.lax.GatherScatterMode.PROMISE_IN_BOUNDS)
                    best = jnp.maximum(best, shuf)
                for i in range(nchunk):
                    sl = pl.ds(i * 16, 16)
                    chunk = row_v[sl]
                    row_v[sl] = jnp.where(chunk == best, neg, chunk)
                return carry

            jax.lax.fori_loop(0, u_q, body, jnp.int32(0))
            one = jnp.full((16,), 1.0, jnp.float32)
            zero = jnp.zeros((16,), jnp.float32)
            for i in range(nchunk):
                sl = pl.ds(i * 16, 16)
                sel_v[sl] = jnp.where(row_v[sl] == neg, one, zero)
            pltpu.sync_copy(sel_v, sel_hbm.at[wid])

    return sc_topk


def _stage_b_kernel(sel_ref, vals_ref, csum_ref, xq_ref, wfc_ref,
                    g_ref, b_ref, o_ref):
    f32 = jnp.float32
    acc = None
    for h in range(H):
        sel_h = sel_ref[h:h + 1].T > f32(0.5)
        ctx = jnp.where(sel_h, vals_ref[0, h], csum_ref[0, h])
        part = jnp.dot(ctx, wfc_ref[h], preferred_element_type=f32)
        acc = part if acc is None else acc + part
    x = acc + xq_ref[0]
    mu = jnp.mean(x, axis=1, keepdims=True)
    xc = x - mu
    var = jnp.mean(xc * xc, axis=1, keepdims=True)
    o_ref[0] = xc * jax.lax.rsqrt(var + f32(1e-5)) * g_ref[...] + b_ref[...]


def kernel(input_Q, input_K, input_V, attn_mask, W_Q, W_K, W_V, W_fc,
           ln_gamma, ln_beta):
    B, L_Q, _ = input_Q.shape
    L_K = input_K.shape[1]
    u_k, u_q, cnt_np, tri_np = _constants(L_Q, L_K)
    cnt = jnp.asarray(cnt_np, dtype=jnp.bfloat16)
    tri = jnp.asarray(tri_np, dtype=jnp.bfloat16)

    m_all, vals, csum = pl.pallas_call(
        functools.partial(_stage_a_kernel, u_k, L_Q, L_K),
        grid=(B,),
        in_specs=[
            pl.BlockSpec((1, L_Q, D_MODEL), lambda b: (b, 0, 0)),
            pl.BlockSpec((1, L_K, D_MODEL), lambda b: (b, 0, 0)),
            pl.BlockSpec((1, L_K, D_MODEL), lambda b: (b, 0, 0)),
            pl.BlockSpec((H, D_MODEL, D_K), lambda b: (0, 0, 0)),
            pl.BlockSpec((H, D_MODEL, D_K), lambda b: (0, 0, 0)),
            pl.BlockSpec((H, D_MODEL, D_V), lambda b: (0, 0, 0)),
            pl.BlockSpec((L_Q, L_K), lambda b: (0, 0)),
            pl.BlockSpec((L_K, L_K), lambda b: (0, 0)),
        ],
        out_specs=[
            pl.BlockSpec((H, L_Q), lambda b: (b, 0)),
            pl.BlockSpec((1, H, L_Q, D_V), lambda b: (b, 0, 0, 0)),
            pl.BlockSpec((1, H, L_K, D_V), lambda b: (b, 0, 0, 0)),
        ],
        out_shape=[
            jax.ShapeDtypeStruct((B * H, L_Q), jnp.float32),
            jax.ShapeDtypeStruct((B, H, L_Q, D_V), jnp.float32),
            jax.ShapeDtypeStruct((B, H, L_K, D_V), jnp.float32),
        ],
        compiler_params=pltpu.CompilerParams(
            dimension_semantics=("parallel",)),
    )(input_Q, input_K, input_V,
      W_Q.reshape(D_MODEL, H, D_K).transpose(1, 0, 2),
      W_K.reshape(D_MODEL, H, D_K).transpose(1, 0, 2),
      W_V.reshape(D_MODEL, H, D_V).transpose(1, 0, 2),
      cnt, tri)

    sel = _make_sc_topk(u_q, B * H, L_Q)(m_all)

    out = pl.pallas_call(
        _stage_b_kernel,
        grid=(B,),
        in_specs=[
            pl.BlockSpec((H, L_Q), lambda b: (b, 0)),
            pl.BlockSpec((1, H, L_Q, D_V), lambda b: (b, 0, 0, 0)),
            pl.BlockSpec((1, H, L_K, D_V), lambda b: (b, 0, 0, 0)),
            pl.BlockSpec((1, L_Q, D_MODEL), lambda b: (b, 0, 0)),
            pl.BlockSpec((H, D_V, D_MODEL), lambda b: (0, 0, 0)),
            pl.BlockSpec((1, D_MODEL), lambda b: (0, 0)),
            pl.BlockSpec((1, D_MODEL), lambda b: (0, 0)),
        ],
        out_specs=pl.BlockSpec((1, L_Q, D_MODEL), lambda b: (b, 0, 0)),
        out_shape=jax.ShapeDtypeStruct((B, L_Q, D_MODEL), jnp.float32),
        compiler_params=pltpu.CompilerParams(
            dimension_semantics=("parallel",)),
    )(sel, vals, csum, input_Q, W_fc.reshape(H, D_V, D_MODEL),
      ln_gamma.reshape(1, D_MODEL), ln_beta.reshape(1, D_MODEL))
    return out


# speedup vs baseline: 1.2801x; 1.0356x over previous
import functools

import numpy as np
import jax
import jax.numpy as jnp
from jax.experimental import pallas as pl
from jax.experimental.pallas import tpu as pltpu
from jax.experimental.pallas import tpu_sc as plsc

D_MODEL = 512
D_K = 64
D_V = 64
H = 8
_C = 5


def _threefry2x32_np(k1, k2, x1, x2):
    u32 = np.uint32
    def rotl(v, d):
        return ((v << u32(d)) | (v >> u32(32 - d))).astype(u32)
    ks = [u32(k1), u32(k2), u32(k1) ^ u32(k2) ^ u32(0x1BD11BDA)]
    x = [(x1 + ks[0]).astype(u32), (x2 + ks[1]).astype(u32)]
    rotations = ((13, 15, 26, 6), (17, 29, 16, 24))
    for i in range(5):
        for r in rotations[i % 2]:
            x[0] = (x[0] + x[1]).astype(u32)
            x[1] = x[0] ^ rotl(x[1], r)
        x[0] = (x[0] + ks[(i + 1) % 3]).astype(u32)
        x[1] = (x[1] + ks[(i + 2) % 3] + u32(i + 1)).astype(u32)
    return x[0], x[1]


def _randint_np(seed, shape, minval, maxval):
    u32 = np.uint32
    n = int(np.prod(shape))
    b1, b2 = _threefry2x32_np(u32(0), u32(seed),
                              np.zeros(2, u32), np.arange(2, dtype=u32))
    idx = np.arange(n, dtype=np.uint64)
    hi = (idx >> np.uint64(32)).astype(u32)
    lo = idx.astype(u32)

    def rbits(ka, kb):
        a, b = _threefry2x32_np(ka, kb, hi, lo)
        return a ^ b

    higher_bits = rbits(b1[0], b2[0])
    lower_bits = rbits(b1[1], b2[1])
    span = u32(maxval - minval)
    multiplier = u32((2 ** 16) % int(span))
    multiplier = u32((int(multiplier) * int(multiplier)) % int(span))
    with np.errstate(over='ignore'):
        offset = ((higher_bits % span) * multiplier + (lower_bits % span)) % span
    return (np.int32(minval) + offset.astype(np.int32)).reshape(shape)


@functools.lru_cache(maxsize=None)
def _constants(L_Q, L_K):
    u_k = min(int(_C * np.log(L_K)), L_Q)
    u_q = min(int(_C * np.log(L_Q)), L_Q)
    idx = _randint_np(42, (L_Q, u_k), 0, L_K)
    cnt = np.zeros((L_Q, L_K), np.float32)
    np.add.at(cnt, (np.arange(L_Q)[:, None], idx), 1.0)
    tri = np.tril(np.ones((L_K, L_K), np.float32))
    return u_k, u_q, cnt, tri


def _stage_a_kernel(u_k, L_Q, L_K,
                    xq_ref, xk_ref, xv_ref, wq_ref, wk_ref, wv_ref,
                    cnt_ref, tri_ref, m_ref, vals_ref, csum_ref):
    f32 = jnp.float32
    neg = f32(-jnp.inf)
    xq = xq_ref[0]
    xk = xk_ref[0]
    xv = xv_ref[0]
    cnt = cnt_ref[...].astype(f32)
    supported = cnt > f32(0)
    tri = tri_ref[...].astype(f32)

    for h in range(H):
        Qh = jnp.dot(xq, wq_ref[h], preferred_element_type=f32)
        Kh = jnp.dot(xk, wk_ref[h], preferred_element_type=f32)
        Vh = jnp.dot(xv, wv_ref[h], preferred_element_type=f32)
        S = jnp.dot(Qh, Kh.T, preferred_element_type=f32)
        m_max = jnp.max(jnp.where(supported, S, neg), axis=1, keepdims=True)
        m_mean = jnp.sum(S * cnt, axis=1, keepdims=True) * f32(1.0 / u_k)
        m_ref[h:h + 1, :] = (m_max - m_mean).T
        Ss = S * f32(1.0 / np.sqrt(D_K))
        e = jnp.exp(Ss - jnp.max(Ss, axis=1, keepdims=True))
        r = f32(1.0) / jnp.sum(e, axis=1, keepdims=True)
        vals = jnp.dot(e, Vh, preferred_element_type=f32) * r
        vals_ref[0, h] = vals.astype(jnp.bfloat16)
        csum = jnp.dot(tri, Vh, preferred_element_type=f32)
        csum_ref[0, h] = csum.astype(jnp.bfloat16)


def _make_sc_topk(u_q, R, L):
    info = plsc.get_sparse_core_info()
    nc = info.num_cores
    mesh = plsc.VectorSubcoreMesh(core_axis_name="c", subcore_axis_name="s")

    nchunk = L // 16

    @functools.partial(
        pl.kernel, mesh=mesh,
        out_type=jax.ShapeDtypeStruct((R, L), jnp.float32),
        scratch_types=[
            pltpu.VMEM((L,), jnp.float32),
            pltpu.VMEM((L,), jnp.float32),
        ],
    )
    def sc_topk(m_hbm, sel_hbm, row_v, sel_v):
        wid = jax.lax.axis_index("s") * nc + jax.lax.axis_index("c")

        @pl.when(wid < R)
        def _():
            pltpu.sync_copy(m_hbm.at[wid], row_v)
            neg = jnp.float32(-jnp.inf)
            iota = jax.lax.iota(jnp.int32, 16)
            perms = [iota ^ k for k in (1, 2, 4, 8)]

            def body(_, carry):
                best = row_v[pl.ds(0, 16)]
                for i in range(1, nchunk):
                    best = jnp.maximum(best, row_v[pl.ds(i * 16, 16)])
                dnums = jax.lax.GatherDimensionNumbers(
                    offset_dims=(), collapsed_slice_dims=(0,),
                    start_index_map=(0,))
                for p in perms:
                    shuf = jax.lax.gather(
                        best, p[:, None], dnums, slice_sizes=(1,),
                        mode=jax.lax.GatherScatterMode.PROMISE_IN_BOUNDS)
                    best = jnp.maximum(best, shuf)
                for i in range(nchunk):
                    sl = pl.ds(i * 16, 16)
                    chunk = row_v[sl]
                    row_v[sl] = jnp.where(chunk == best, neg, chunk)
                return carry

            jax.lax.fori_loop(0, u_q, body, jnp.int32(0))
            one = jnp.full((16,), 1.0, jnp.float32)
            zero = jnp.zeros((16,), jnp.float32)
            for i in range(nchunk):
                sl = pl.ds(i * 16, 16)
                sel_v[sl] = jnp.where(row_v[sl] == neg, one, zero)
            pltpu.sync_copy(sel_v, sel_hbm.at[wid])

    return sc_topk


def _stage_b_kernel(sel_ref, vals_ref, csum_ref, xq_ref, wfc_ref,
                    g_ref, b_ref, o_ref):
    f32 = jnp.float32
    acc = None
    for h in range(H):
        sel_h = sel_ref[h:h + 1].T > f32(0.5)
        ctx = jnp.where(sel_h, vals_ref[0, h].astype(f32),
                        csum_ref[0, h].astype(f32))
        part = jnp.dot(ctx, wfc_ref[h], preferred_element_type=f32)
        acc = part if acc is None else acc + part
    x = acc + xq_ref[0]
    mu = jnp.mean(x, axis=1, keepdims=True)
    xc = x - mu
    var = jnp.mean(xc * xc, axis=1, keepdims=True)
    o_ref[0] = xc * jax.lax.rsqrt(var + f32(1e-5)) * g_ref[...] + b_ref[...]


def kernel(input_Q, input_K, input_V, attn_mask, W_Q, W_K, W_V, W_fc,
           ln_gamma, ln_beta):
    B, L_Q, _ = input_Q.shape
    L_K = input_K.shape[1]
    u_k, u_q, cnt_np, tri_np = _constants(L_Q, L_K)
    cnt = jnp.asarray(cnt_np, dtype=jnp.bfloat16)
    tri = jnp.asarray(tri_np, dtype=jnp.bfloat16)

    m_all, vals, csum = pl.pallas_call(
        functools.partial(_stage_a_kernel, u_k, L_Q, L_K),
        grid=(B,),
        in_specs=[
            pl.BlockSpec((1, L_Q, D_MODEL), lambda b: (b, 0, 0)),
            pl.BlockSpec((1, L_K, D_MODEL), lambda b: (b, 0, 0)),
            pl.BlockSpec((1, L_K, D_MODEL), lambda b: (b, 0, 0)),
            pl.BlockSpec((H, D_MODEL, D_K), lambda b: (0, 0, 0)),
            pl.BlockSpec((H, D_MODEL, D_K), lambda b: (0, 0, 0)),
            pl.BlockSpec((H, D_MODEL, D_V), lambda b: (0, 0, 0)),
            pl.BlockSpec((L_Q, L_K), lambda b: (0, 0)),
            pl.BlockSpec((L_K, L_K), lambda b: (0, 0)),
        ],
        out_specs=[
            pl.BlockSpec((H, L_Q), lambda b: (b, 0)),
            pl.BlockSpec((1, H, L_Q, D_V), lambda b: (b, 0, 0, 0)),
            pl.BlockSpec((1, H, L_K, D_V), lambda b: (b, 0, 0, 0)),
        ],
        out_shape=[
            jax.ShapeDtypeStruct((B * H, L_Q), jnp.float32),
            jax.ShapeDtypeStruct((B, H, L_Q, D_V), jnp.bfloat16),
            jax.ShapeDtypeStruct((B, H, L_K, D_V), jnp.bfloat16),
        ],
        compiler_params=pltpu.CompilerParams(
            dimension_semantics=("parallel",)),
    )(input_Q, input_K, input_V,
      W_Q.reshape(D_MODEL, H, D_K).transpose(1, 0, 2),
      W_K.reshape(D_MODEL, H, D_K).transpose(1, 0, 2),
      W_V.reshape(D_MODEL, H, D_V).transpose(1, 0, 2),
      cnt, tri)

    sel = _make_sc_topk(u_q, B * H, L_Q)(m_all)

    out = pl.pallas_call(
        _stage_b_kernel,
        grid=(B,),
        in_specs=[
            pl.BlockSpec((H, L_Q), lambda b: (b, 0)),
            pl.BlockSpec((1, H, L_Q, D_V), lambda b: (b, 0, 0, 0)),
            pl.BlockSpec((1, H, L_K, D_V), lambda b: (b, 0, 0, 0)),
            pl.BlockSpec((1, L_Q, D_MODEL), lambda b: (b, 0, 0)),
            pl.BlockSpec((H, D_V, D_MODEL), lambda b: (0, 0, 0)),
            pl.BlockSpec((1, D_MODEL), lambda b: (0, 0)),
            pl.BlockSpec((1, D_MODEL), lambda b: (0, 0)),
        ],
        out_specs=pl.BlockSpec((1, L_Q, D_MODEL), lambda b: (b, 0, 0)),
        out_shape=jax.ShapeDtypeStruct((B, L_Q, D_MODEL), jnp.float32),
        compiler_params=pltpu.CompilerParams(
            dimension_semantics=("parallel",)),
    )(sel, vals, csum, input_Q, W_fc.reshape(H, D_V, D_MODEL),
      ln_gamma.reshape(1, D_MODEL), ln_beta.reshape(1, D_MODEL))
    return out


# speedup vs baseline: 1.6510x; 1.2898x over previous
import functools

import numpy as np
import jax
import jax.numpy as jnp
from jax.experimental import pallas as pl
from jax.experimental.pallas import tpu as pltpu
from jax.experimental.pallas import tpu_sc as plsc

D_MODEL = 512
D_K = 64
D_V = 64
H = 8
_C = 5


def _threefry2x32_np(k1, k2, x1, x2):
    u32 = np.uint32
    def rotl(v, d):
        return ((v << u32(d)) | (v >> u32(32 - d))).astype(u32)
    ks = [u32(k1), u32(k2), u32(k1) ^ u32(k2) ^ u32(0x1BD11BDA)]
    x = [(x1 + ks[0]).astype(u32), (x2 + ks[1]).astype(u32)]
    rotations = ((13, 15, 26, 6), (17, 29, 16, 24))
    for i in range(5):
        for r in rotations[i % 2]:
            x[0] = (x[0] + x[1]).astype(u32)
            x[1] = x[0] ^ rotl(x[1], r)
        x[0] = (x[0] + ks[(i + 1) % 3]).astype(u32)
        x[1] = (x[1] + ks[(i + 2) % 3] + u32(i + 1)).astype(u32)
    return x[0], x[1]


def _randint_np(seed, shape, minval, maxval):
    u32 = np.uint32
    n = int(np.prod(shape))
    b1, b2 = _threefry2x32_np(u32(0), u32(seed),
                              np.zeros(2, u32), np.arange(2, dtype=u32))
    idx = np.arange(n, dtype=np.uint64)
    hi = (idx >> np.uint64(32)).astype(u32)
    lo = idx.astype(u32)

    def rbits(ka, kb):
        a, b = _threefry2x32_np(ka, kb, hi, lo)
        return a ^ b

    higher_bits = rbits(b1[0], b2[0])
    lower_bits = rbits(b1[1], b2[1])
    span = u32(maxval - minval)
    multiplier = u32((2 ** 16) % int(span))
    multiplier = u32((int(multiplier) * int(multiplier)) % int(span))
    with np.errstate(over='ignore'):
        offset = ((higher_bits % span) * multiplier + (lower_bits % span)) % span
    return (np.int32(minval) + offset.astype(np.int32)).reshape(shape)


@functools.lru_cache(maxsize=None)
def _constants(L_Q, L_K):
    u_k = min(int(_C * np.log(L_K)), L_Q)
    u_q = min(int(_C * np.log(L_Q)), L_Q)
    idx = _randint_np(42, (L_Q, u_k), 0, L_K)
    cnt = np.zeros((L_Q, L_K), np.float32)
    np.add.at(cnt, (np.arange(L_Q)[:, None], idx), 1.0)
    tri = np.tril(np.ones((L_K, L_K), np.float32))
    return u_k, u_q, cnt, tri


def _stage_a_kernel(u_k, L_Q, L_K,
                    xq_ref, xk_ref, xv_ref, wq_ref, wk_ref, wv_ref,
                    cnt_ref, tri_ref, m_ref, vals_ref, csum_ref):
    f32 = jnp.float32
    neg = f32(-jnp.inf)
    xq = xq_ref[0]
    xk = xk_ref[0]
    xv = xv_ref[0]
    cnt = cnt_ref[...].astype(f32)
    supported = cnt > f32(0)
    tri = tri_ref[...].astype(f32)

    Q = jnp.dot(xq, wq_ref[...], preferred_element_type=f32)
    K = jnp.dot(xk, wk_ref[...], preferred_element_type=f32)
    V = jnp.dot(xv, wv_ref[...], preferred_element_type=f32)
    for h in range(H):
        Qh = Q[:, h * D_K:(h + 1) * D_K]
        Kh = K[:, h * D_K:(h + 1) * D_K]
        Vh = V[:, h * D_V:(h + 1) * D_V]
        S = jnp.dot(Qh, Kh.T, preferred_element_type=f32)
        m_max = jnp.max(jnp.where(supported, S, neg), axis=1, keepdims=True)
        m_mean = jnp.sum(S * cnt, axis=1, keepdims=True) * f32(1.0 / u_k)
        m_ref[h:h + 1, :] = (m_max - m_mean).T
        Ss = S * f32(1.0 / np.sqrt(D_K))
        e = jnp.exp(Ss - jnp.max(Ss, axis=1, keepdims=True))
        r = f32(1.0) / jnp.sum(e, axis=1, keepdims=True)
        vals = jnp.dot(e, Vh, preferred_element_type=f32) * r
        vals_ref[0, h] = vals.astype(jnp.bfloat16)
        csum = jnp.dot(tri, Vh, preferred_element_type=f32)
        csum_ref[0, h] = csum.astype(jnp.bfloat16)


def _make_sc_topk(u_q, R, L):
    info = plsc.get_sparse_core_info()
    nc = info.num_cores
    mesh = plsc.VectorSubcoreMesh(core_axis_name="c", subcore_axis_name="s")

    nchunk = L // 16

    @functools.partial(
        pl.kernel, mesh=mesh,
        out_type=jax.ShapeDtypeStruct((R, L), jnp.float32),
        scratch_types=[
            pltpu.VMEM((L,), jnp.float32),
            pltpu.VMEM((L,), jnp.float32),
        ],
    )
    def sc_topk(m_hbm, sel_hbm, row_v, sel_v):
        wid = jax.lax.axis_index("s") * nc + jax.lax.axis_index("c")

        @pl.when(wid < R)
        def _():
            pltpu.sync_copy(m_hbm.at[wid], row_v)
            neg = jnp.float32(-jnp.inf)
            iota = jax.lax.iota(jnp.int32, 16)
            perms = [iota ^ k for k in (1, 2, 4, 8)]

            def body(_, carry):
                best = row_v[pl.ds(0, 16)]
                for i in range(1, nchunk):
                    best = jnp.maximum(best, row_v[pl.ds(i * 16, 16)])
                dnums = jax.lax.GatherDimensionNumbers(
                    offset_dims=(), collapsed_slice_dims=(0,),
                    start_index_map=(0,))
                for p in perms:
                    shuf = jax.lax.gather(
                        best, p[:, None], dnums, slice_sizes=(1,),
                        mode=jax.lax.GatherScatterMode.PROMISE_IN_BOUNDS)
                    best = jnp.maximum(best, shuf)
                for i in range(nchunk):
                    sl = pl.ds(i * 16, 16)
                    chunk = row_v[sl]
                    row_v[sl] = jnp.where(chunk == best, neg, chunk)
                return carry

            jax.lax.fori_loop(0, u_q, body, jnp.int32(0))
            one = jnp.full((16,), 1.0, jnp.float32)
            zero = jnp.zeros((16,), jnp.float32)
            for i in range(nchunk):
                sl = pl.ds(i * 16, 16)
                sel_v[sl] = jnp.where(row_v[sl] == neg, one, zero)
            pltpu.sync_copy(sel_v, sel_hbm.at[wid])

    return sc_topk


def _stage_b_kernel(sel_ref, vals_ref, csum_ref, xq_ref, wfc_ref,
                    g_ref, b_ref, o_ref):
    f32 = jnp.float32
    acc = None
    for h in range(H):
        sel_h = sel_ref[h:h + 1].T > f32(0.5)
        ctx = jnp.where(sel_h, vals_ref[0, h].astype(f32),
                        csum_ref[0, h].astype(f32))
        part = jnp.dot(ctx, wfc_ref[h], preferred_element_type=f32)
        acc = part if acc is None else acc + part
    x = acc + xq_ref[0]
    mu = jnp.mean(x, axis=1, keepdims=True)
    xc = x - mu
    var = jnp.mean(xc * xc, axis=1, keepdims=True)
    o_ref[0] = xc * jax.lax.rsqrt(var + f32(1e-5)) * g_ref[...] + b_ref[...]


def kernel(input_Q, input_K, input_V, attn_mask, W_Q, W_K, W_V, W_fc,
           ln_gamma, ln_beta):
    B, L_Q, _ = input_Q.shape
    L_K = input_K.shape[1]
    u_k, u_q, cnt_np, tri_np = _constants(L_Q, L_K)
    cnt = jnp.asarray(cnt_np, dtype=jnp.bfloat16)
    tri = jnp.asarray(tri_np, dtype=jnp.bfloat16)

    m_all, vals, csum = pl.pallas_call(
        functools.partial(_stage_a_kernel, u_k, L_Q, L_K),
        grid=(B,),
        in_specs=[
            pl.BlockSpec((1, L_Q, D_MODEL), lambda b: (b, 0, 0)),
            pl.BlockSpec((1, L_K, D_MODEL), lambda b: (b, 0, 0)),
            pl.BlockSpec((1, L_K, D_MODEL), lambda b: (b, 0, 0)),
            pl.BlockSpec((D_MODEL, H * D_K), lambda b: (0, 0)),
            pl.BlockSpec((D_MODEL, H * D_K), lambda b: (0, 0)),
            pl.BlockSpec((D_MODEL, H * D_V), lambda b: (0, 0)),
            pl.BlockSpec((L_Q, L_K), lambda b: (0, 0)),
            pl.BlockSpec((L_K, L_K), lambda b: (0, 0)),
        ],
        out_specs=[
            pl.BlockSpec((H, L_Q), lambda b: (b, 0)),
            pl.BlockSpec((1, H, L_Q, D_V), lambda b: (b, 0, 0, 0)),
            pl.BlockSpec((1, H, L_K, D_V), lambda b: (b, 0, 0, 0)),
        ],
        out_shape=[
            jax.ShapeDtypeStruct((B * H, L_Q), jnp.float32),
            jax.ShapeDtypeStruct((B, H, L_Q, D_V), jnp.bfloat16),
            jax.ShapeDtypeStruct((B, H, L_K, D_V), jnp.bfloat16),
        ],
        compiler_params=pltpu.CompilerParams(
            dimension_semantics=("parallel",)),
    )(input_Q, input_K, input_V, W_Q, W_K, W_V, cnt, tri)

    sel = _make_sc_topk(u_q, B * H, L_Q)(m_all)

    out = pl.pallas_call(
        _stage_b_kernel,
        grid=(B,),
        in_specs=[
            pl.BlockSpec((H, L_Q), lambda b: (b, 0)),
            pl.BlockSpec((1, H, L_Q, D_V), lambda b: (b, 0, 0, 0)),
            pl.BlockSpec((1, H, L_K, D_V), lambda b: (b, 0, 0, 0)),
            pl.BlockSpec((1, L_Q, D_MODEL), lambda b: (b, 0, 0)),
            pl.BlockSpec((H, D_V, D_MODEL), lambda b: (0, 0, 0)),
            pl.BlockSpec((1, D_MODEL), lambda b: (0, 0)),
            pl.BlockSpec((1, D_MODEL), lambda b: (0, 0)),
        ],
        out_specs=pl.BlockSpec((1, L_Q, D_MODEL), lambda b: (b, 0, 0)),
        out_shape=jax.ShapeDtypeStruct((B, L_Q, D_MODEL), jnp.float32),
        compiler_params=pltpu.CompilerParams(
            dimension_semantics=("parallel",)),
    )(sel, vals, csum, input_Q, W_fc.reshape(H, D_V, D_MODEL),
      ln_gamma.reshape(1, D_MODEL), ln_beta.reshape(1, D_MODEL))
    return out


# speedup vs baseline: 1.6629x; 1.0072x over previous
import functools

import numpy as np
import jax
import jax.numpy as jnp
from jax.experimental import pallas as pl
from jax.experimental.pallas import tpu as pltpu
from jax.experimental.pallas import tpu_sc as plsc

D_MODEL = 512
D_K = 64
D_V = 64
H = 8
_C = 5


def _threefry2x32_np(k1, k2, x1, x2):
    u32 = np.uint32
    def rotl(v, d):
        return ((v << u32(d)) | (v >> u32(32 - d))).astype(u32)
    ks = [u32(k1), u32(k2), u32(k1) ^ u32(k2) ^ u32(0x1BD11BDA)]
    x = [(x1 + ks[0]).astype(u32), (x2 + ks[1]).astype(u32)]
    rotations = ((13, 15, 26, 6), (17, 29, 16, 24))
    for i in range(5):
        for r in rotations[i % 2]:
            x[0] = (x[0] + x[1]).astype(u32)
            x[1] = x[0] ^ rotl(x[1], r)
        x[0] = (x[0] + ks[(i + 1) % 3]).astype(u32)
        x[1] = (x[1] + ks[(i + 2) % 3] + u32(i + 1)).astype(u32)
    return x[0], x[1]


def _randint_np(seed, shape, minval, maxval):
    u32 = np.uint32
    n = int(np.prod(shape))
    b1, b2 = _threefry2x32_np(u32(0), u32(seed),
                              np.zeros(2, u32), np.arange(2, dtype=u32))
    idx = np.arange(n, dtype=np.uint64)
    hi = (idx >> np.uint64(32)).astype(u32)
    lo = idx.astype(u32)

    def rbits(ka, kb):
        a, b = _threefry2x32_np(ka, kb, hi, lo)
        return a ^ b

    higher_bits = rbits(b1[0], b2[0])
    lower_bits = rbits(b1[1], b2[1])
    span = u32(maxval - minval)
    multiplier = u32((2 ** 16) % int(span))
    multiplier = u32((int(multiplier) * int(multiplier)) % int(span))
    with np.errstate(over='ignore'):
        offset = ((higher_bits % span) * multiplier + (lower_bits % span)) % span
    return (np.int32(minval) + offset.astype(np.int32)).reshape(shape)


@functools.lru_cache(maxsize=None)
def _constants(L_Q, L_K):
    u_k = min(int(_C * np.log(L_K)), L_Q)
    u_q = min(int(_C * np.log(L_Q)), L_Q)
    idx = _randint_np(42, (L_Q, u_k), 0, L_K)
    cnt = np.zeros((L_Q, L_K), np.float32)
    np.add.at(cnt, (np.arange(L_Q)[:, None], idx), 1.0)
    tri = np.tril(np.ones((L_K, L_K), np.float32))
    return u_k, u_q, cnt, tri


def _stage_a_kernel(u_k, L_Q, L_K,
                    xq_ref, xk_ref, xv_ref, wq_ref, wk_ref, wv_ref,
                    cnt_ref, tri_ref, m_ref, vals_ref, csum_ref):
    f32 = jnp.float32
    neg = f32(-jnp.inf)
    xq = xq_ref[0]
    xk = xk_ref[0]
    xv = xv_ref[0]
    cnt = cnt_ref[...].astype(f32)
    supported = cnt > f32(0)
    tri = tri_ref[...].astype(f32)

    Q = jnp.dot(xq, wq_ref[...], preferred_element_type=f32)
    K = jnp.dot(xk, wk_ref[...], preferred_element_type=f32)
    V = jnp.dot(xv, wv_ref[...], preferred_element_type=f32)
    for h in range(H):
        Qh = Q[:, h * D_K:(h + 1) * D_K]
        Kh = K[:, h * D_K:(h + 1) * D_K]
        Vh = V[:, h * D_V:(h + 1) * D_V]
        S = jnp.dot(Qh, Kh.T, preferred_element_type=f32)
        m_max = jnp.max(jnp.where(supported, S, neg), axis=1, keepdims=True)
        m_mean = jnp.sum(S * cnt, axis=1, keepdims=True) * f32(1.0 / u_k)
        m_ref[h:h + 1, :] = (m_max - m_mean).T
        Ss = S * f32(1.0 / np.sqrt(D_K))
        e = jnp.exp(Ss - jnp.max(Ss, axis=1, keepdims=True))
        r = f32(1.0) / jnp.sum(e, axis=1, keepdims=True)
        vals = jnp.dot(e, Vh, preferred_element_type=f32) * r
        vals_ref[0, h] = vals.astype(jnp.bfloat16)
        csum = jnp.dot(tri, Vh, preferred_element_type=f32)
        csum_ref[0, h] = csum.astype(jnp.bfloat16)


def _make_sc_topk(u_q, R, L):
    info = plsc.get_sparse_core_info()
    nc = info.num_cores
    mesh = plsc.VectorSubcoreMesh(core_axis_name="c", subcore_axis_name="s")

    nchunk = L // 16

    @functools.partial(
        pl.kernel, mesh=mesh,
        out_type=jax.ShapeDtypeStruct((R, L), jnp.float32),
        scratch_types=[
            pltpu.VMEM((L,), jnp.float32),
            pltpu.VMEM((L,), jnp.float32),
        ],
    )
    def sc_topk(m_hbm, sel_hbm, row_v, sel_v):
        wid = jax.lax.axis_index("s") * nc + jax.lax.axis_index("c")

        @pl.when(wid < R)
        def _():
            pltpu.sync_copy(m_hbm.at[wid], row_v)
            neg = jnp.float32(-jnp.inf)
            iota = jax.lax.iota(jnp.int32, 16)
            perms = [iota ^ k for k in (1, 2, 4, 8)]

            def body(_, carry):
                best = row_v[pl.ds(0, 16)]
                for i in range(1, nchunk):
                    best = jnp.maximum(best, row_v[pl.ds(i * 16, 16)])
                dnums = jax.lax.GatherDimensionNumbers(
                    offset_dims=(), collapsed_slice_dims=(0,),
                    start_index_map=(0,))
                for p in perms:
                    shuf = jax.lax.gather(
                        best, p[:, None], dnums, slice_sizes=(1,),
                        mode=jax.lax.GatherScatterMode.PROMISE_IN_BOUNDS)
                    best = jnp.maximum(best, shuf)
                for i in range(nchunk):
                    sl = pl.ds(i * 16, 16)
                    chunk = row_v[sl]
                    row_v[sl] = jnp.where(chunk == best, neg, chunk)
                return carry

            jax.lax.fori_loop(0, u_q, body, jnp.int32(0))
            one = jnp.full((16,), 1.0, jnp.float32)
            zero = jnp.zeros((16,), jnp.float32)
            for i in range(nchunk):
                sl = pl.ds(i * 16, 16)
                sel_v[sl] = jnp.where(row_v[sl] == neg, one, zero)
            pltpu.sync_copy(sel_v, sel_hbm.at[wid])

    return sc_topk


def _stage_b_kernel(sel_ref, vals_ref, csum_ref, xq_ref, wfc_ref,
                    g_ref, b_ref, o_ref):
    f32 = jnp.float32
    ctxs = []
    for h in range(H):
        sel_h = sel_ref[h:h + 1].T > f32(0.5)
        ctxs.append(jnp.where(sel_h, vals_ref[0, h].astype(f32),
                              csum_ref[0, h].astype(f32)))
    ctx = jnp.concatenate(ctxs, axis=1)
    acc = jnp.dot(ctx, wfc_ref[...], preferred_element_type=f32)
    x = acc + xq_ref[0]
    mu = jnp.mean(x, axis=1, keepdims=True)
    xc = x - mu
    var = jnp.mean(xc * xc, axis=1, keepdims=True)
    o_ref[0] = xc * jax.lax.rsqrt(var + f32(1e-5)) * g_ref[...] + b_ref[...]


def kernel(input_Q, input_K, input_V, attn_mask, W_Q, W_K, W_V, W_fc,
           ln_gamma, ln_beta):
    B, L_Q, _ = input_Q.shape
    L_K = input_K.shape[1]
    u_k, u_q, cnt_np, tri_np = _constants(L_Q, L_K)
    cnt = jnp.asarray(cnt_np, dtype=jnp.bfloat16)
    tri = jnp.asarray(tri_np, dtype=jnp.bfloat16)

    m_all, vals, csum = pl.pallas_call(
        functools.partial(_stage_a_kernel, u_k, L_Q, L_K),
        grid=(B,),
        in_specs=[
            pl.BlockSpec((1, L_Q, D_MODEL), lambda b: (b, 0, 0)),
            pl.BlockSpec((1, L_K, D_MODEL), lambda b: (b, 0, 0)),
            pl.BlockSpec((1, L_K, D_MODEL), lambda b: (b, 0, 0)),
            pl.BlockSpec((D_MODEL, H * D_K), lambda b: (0, 0)),
            pl.BlockSpec((D_MODEL, H * D_K), lambda b: (0, 0)),
            pl.BlockSpec((D_MODEL, H * D_V), lambda b: (0, 0)),
            pl.BlockSpec((L_Q, L_K), lambda b: (0, 0)),
            pl.BlockSpec((L_K, L_K), lambda b: (0, 0)),
        ],
        out_specs=[
            pl.BlockSpec((H, L_Q), lambda b: (b, 0)),
            pl.BlockSpec((1, H, L_Q, D_V), lambda b: (b, 0, 0, 0)),
            pl.BlockSpec((1, H, L_K, D_V), lambda b: (b, 0, 0, 0)),
        ],
        out_shape=[
            jax.ShapeDtypeStruct((B * H, L_Q), jnp.float32),
            jax.ShapeDtypeStruct((B, H, L_Q, D_V), jnp.bfloat16),
            jax.ShapeDtypeStruct((B, H, L_K, D_V), jnp.bfloat16),
        ],
        compiler_params=pltpu.CompilerParams(
            dimension_semantics=("parallel",)),
    )(input_Q, input_K, input_V, W_Q, W_K, W_V, cnt, tri)

    sel = _make_sc_topk(u_q, B * H, L_Q)(m_all)

    out = pl.pallas_call(
        _stage_b_kernel,
        grid=(B,),
        in_specs=[
            pl.BlockSpec((H, L_Q), lambda b: (b, 0)),
            pl.BlockSpec((1, H, L_Q, D_V), lambda b: (b, 0, 0, 0)),
            pl.BlockSpec((1, H, L_K, D_V), lambda b: (b, 0, 0, 0)),
            pl.BlockSpec((1, L_Q, D_MODEL), lambda b: (b, 0, 0)),
            pl.BlockSpec((H * D_V, D_MODEL), lambda b: (0, 0)),
            pl.BlockSpec((1, D_MODEL), lambda b: (0, 0)),
            pl.BlockSpec((1, D_MODEL), lambda b: (0, 0)),
        ],
        out_specs=pl.BlockSpec((1, L_Q, D_MODEL), lambda b: (b, 0, 0)),
        out_shape=jax.ShapeDtypeStruct((B, L_Q, D_MODEL), jnp.float32),
        compiler_params=pltpu.CompilerParams(
            dimension_semantics=("parallel",)),
    )(sel, vals, csum, input_Q, W_fc,
      ln_gamma.reshape(1, D_MODEL), ln_beta.reshape(1, D_MODEL))
    return out
